# Initial kernel scaffold; baseline (speedup 1.0000x reference)
import jax
import jax.numpy as jnp
from jax.experimental import pallas as pl


def kernel(beta, A, Z, Gate, sample_idx, sparse_sample_i, sparse_sample_j):
    raise NotImplementedError("write your pallas kernel here")


# speedup vs baseline: 3.2905x; 3.2905x over previous
import functools

import jax
import jax.numpy as jnp
from jax import lax
from jax.experimental import pallas as pl
from jax.experimental.pallas import tpu as pltpu

N = 50000
K = 8
D = 2
S = 2500
SP = 2560
ES = 16384
TI = 256

_F32 = jnp.float32
_HIGH = lax.Precision.HIGHEST


def _softmax0(x):
    m = jnp.max(x, axis=0, keepdims=True)
    e = jnp.exp(x - m)
    return e / jnp.sum(e, axis=0, keepdims=True)


def _tc_body(z_ref, gt_ref, a_ref, s1t_ref, s2t_ref, eit_ref, ejt_ref,
             out_ref, xb_ref):
    zs_full = _softmax0(z_ref[...])
    zg = zs_full * jax.nn.sigmoid(gt_ref[...])
    denom = jnp.sum(zg, axis=1, keepdims=True)

    s1t = s1t_ref[...]
    zs_l = _softmax0(s1t[0:K, :])
    beta_l = s1t[K:K + 1, :]
    g_l = jax.nn.sigmoid(s2t_ref[...][0:K, :])
    cid = lax.broadcasted_iota(jnp.int32, (1, SP), 1)
    col_valid = cid < S
    c_l = jnp.where(col_valid, zs_l * g_l / denom, 0.0)

    b_kk = lax.dot_general(zs_l, c_l, (((1,), (1,)), ((), ())),
                           preferred_element_type=_F32, precision=_HIGH)
    azc = lax.dot_general(a_ref[...], b_kk, (((1,), (0,)), ((), ())),
                          preferred_element_type=_F32, precision=_HIGH)
    x_l = lax.dot_general(azc, zs_l, (((1,), (0,)), ((), ())),
                          preferred_element_type=_F32, precision=_HIGH)

    xbt = jnp.concatenate([x_l, beta_l], axis=0)
    xb_ref[...] = xbt.T
    x0l = x_l[0:1, :]
    x1l = x_l[1:2, :]

    def body(t, acc):
        i0 = t * TI
        tile = xb_ref[pl.ds(i0, TI), :]
        xi0 = tile[:, 0:1]
        xi1 = tile[:, 1:2]
        bi = tile[:, 2:3]
        rid = i0 + lax.broadcasted_iota(jnp.int32, (TI, 1), 0)
        d0 = xi0 - x0l + 1e-6
        d1 = xi1 - x1l + 1e-6
        dist = jnp.sqrt(d0 * d0 + d1 * d1)
        m = jnp.exp(bi + beta_l - dist)
        mask = (rid != cid) & (rid < S) & col_valid
        return acc + jnp.sum(jnp.where(mask, m, 0.0))

    tot = lax.fori_loop(0, SP // TI, body, _F32(0.0))
    e1 = jnp.exp(_F32(1.0))
    z1 = 0.5 * e1 * e1 * tot

    eit = eit_ref[...]
    ejt = ejt_ref[...]
    zi = _softmax0(eit[0:K, :])
    zj = _softmax0(ejt[0:K, :])
    pi = lax.dot_general(azc, zi, (((1,), (0,)), ((), ())),
                         preferred_element_type=_F32, precision=_HIGH)
    pj = lax.dot_general(azc, zj, (((1,), (0,)), ((), ())),
                         preferred_element_type=_F32, precision=_HIGH)
    df = pi - pj + 1e-6
    nrm = jnp.sqrt(df[0:1, :] ** 2 + df[1:2, :] ** 2)
    z2 = jnp.sum(eit[K:K + 1, :] + ejt[K:K + 1, :] - nrm)

    out_ref[...] = (z2 - z1)[None, None]


def _gather_rows(t1, t2, sidx, si, sj):
    return t1[sidx], t2[sidx], t1[si], t1[sj]


def _tc_call(Z, gate_t, A, s1t, s2t, eit, ejt):
    return pl.pallas_call(
        _tc_body,
        out_shape=jax.ShapeDtypeStruct((1, 1), _F32),
        scratch_shapes=[pltpu.VMEM((SP, 3), _F32)],
    )(Z, gate_t, A, s1t, s2t, eit, ejt)


def kernel(beta, A, Z, Gate, sample_idx, sparse_sample_i, sparse_sample_j):
    beta = beta.astype(_F32)
    t1 = jnp.concatenate(
        [Z.T, beta[:, None], jnp.zeros((N, 16 - K - 1), _F32)], axis=1)
    t2 = jnp.concatenate([Gate, jnp.zeros((N, 16 - K), _F32)], axis=1)
    sidx = jnp.concatenate(
        [sample_idx.astype(jnp.int32), jnp.zeros((SP - S,), jnp.int32)])
    si = sparse_sample_i.astype(jnp.int32)
    sj = sparse_sample_j.astype(jnp.int32)
    s1, s2, ei, ej = _gather_rows(t1, t2, sidx, si, sj)
    return _tc_call(Z, Gate.T, A, s1.T, s2.T, ei.T, ej.T)
